# Initial kernel scaffold; baseline (speedup 1.0000x reference)
import jax
import jax.numpy as jnp
from jax import lax
from jax.experimental import pallas as pl
from jax.experimental.pallas import tpu as pltpu
from jax.experimental.pallas import tpu_sc as plsc

_VOCAB = 1000000
_EMBED_DIM = 32
_BATCH = 4096
_SEQ_LEN = 200

_NC = 2
_NS = 16
_NW = _NC * _NS

_N = _BATCH * _SEQ_LEN
_IDX_MINOR = 128
_ROWS = 8
_CHUNK = _ROWS * _IDX_MINOR
_PER_W = _N // _NW
_N_CHUNKS = _PER_W // _CHUNK
_MAJOR = _N // _IDX_MINOR


def _body(xt_hbm, table_hbm, out_hbm, idx_v, rows_v, sem):
    wid = lax.axis_index("s") * _NC + lax.axis_index("c")
    row_base = wid * (_PER_W // _IDX_MINOR)

    @pl.loop(0, _N_CHUNKS)
    def _chunk(g):
        row0 = row_base + g * _ROWS
        pltpu.sync_copy(xt_hbm.at[pl.ds(row0, _ROWS)], idx_v)
        copies = []
        for j in range(_ROWS):
            copies.append(
                pltpu.async_copy(table_hbm.at[idx_v.at[j]], rows_v.at[j], sem)
            )
        for c in copies:
            c.wait()
        pltpu.sync_copy(rows_v, out_hbm.at[pl.ds(row0, _ROWS)])


@jax.jit
def kernel(x, table):
    xt = jnp.transpose(x).reshape(_MAJOR, _IDX_MINOR).astype(jnp.int32)

    mesh = plsc.VectorSubcoreMesh(
        core_axis_name="c", subcore_axis_name="s",
        num_cores=_NC, num_subcores=_NS,
    )
    out = pl.kernel(
        _body,
        out_type=jax.ShapeDtypeStruct((_MAJOR, _IDX_MINOR, _EMBED_DIM), jnp.float32),
        mesh=mesh,
        scratch_types=[
            pltpu.VMEM((_ROWS, _IDX_MINOR), jnp.int32),
            pltpu.VMEM((_ROWS, _IDX_MINOR, _EMBED_DIM), jnp.float32),
            pltpu.SemaphoreType.DMA,
        ],
    )(xt, table)
    return out.reshape(_SEQ_LEN, _BATCH, _EMBED_DIM)


# speedup vs baseline: 1.5311x; 1.5311x over previous
import jax
import jax.numpy as jnp
from jax import lax
from jax.experimental import pallas as pl
from jax.experimental.pallas import tpu as pltpu
from jax.experimental.pallas import tpu_sc as plsc

_VOCAB = 1000000
_EMBED_DIM = 32
_BATCH = 4096
_SEQ_LEN = 200

_NC = 2
_NS = 16
_NW = _NC * _NS

_N = _BATCH * _SEQ_LEN
_IDX_MINOR = 128
_ROWS = 8
_CHUNK = _ROWS * _IDX_MINOR
_PER_W = _N // _NW
_N_CHUNKS = _PER_W // _CHUNK
_MAJOR = _N // _IDX_MINOR


def _body(xt_hbm, table_hbm, out_hbm, idx_v, rows_v, sem):
    wid = lax.axis_index("s") * _NC + lax.axis_index("c")
    row_base = wid * (_PER_W // _IDX_MINOR)

    @pl.loop(0, _N_CHUNKS)
    def _chunk(g):
        row0 = row_base + g * _ROWS
        pltpu.sync_copy(xt_hbm.at[pl.ds(row0, _ROWS)], idx_v)
        copies = []
        for j in range(_ROWS):
            copies.append(
                pltpu.async_copy(table_hbm.at[idx_v.at[j]], rows_v.at[j], sem)
            )
        for c in copies:
            c.wait()
        pltpu.sync_copy(rows_v, out_hbm.at[pl.ds(row0, _ROWS)])


@jax.jit
def kernel(x, table):
    xt = jnp.transpose(x).reshape(_MAJOR, _IDX_MINOR).astype(jnp.int32)

    mesh = plsc.VectorSubcoreMesh(
        core_axis_name="c", subcore_axis_name="s",
        num_cores=_NC, num_subcores=_NS,
    )
    out = pl.kernel(
        _body,
        out_type=jax.ShapeDtypeStruct((_MAJOR, _IDX_MINOR, _EMBED_DIM), jnp.float32),
        mesh=mesh,
        scratch_types=[
            pltpu.VMEM((_ROWS, _IDX_MINOR), jnp.int32),
            pltpu.VMEM((_ROWS, _IDX_MINOR, _EMBED_DIM), jnp.float32),
            pltpu.SemaphoreType.DMA,
        ],
        compiler_params=pltpu.CompilerParams(use_tc_tiling_on_sc=False),
    )(xt, table)
    return out.reshape(_SEQ_LEN, _BATCH, _EMBED_DIM)


# speedup vs baseline: 1.5761x; 1.0294x over previous
import jax
import jax.numpy as jnp
from jax import lax
from jax.experimental import pallas as pl
from jax.experimental.pallas import tpu as pltpu
from jax.experimental.pallas import tpu_sc as plsc

_EMBED_DIM = 32
_BATCH = 4096
_SEQ_LEN = 200

_NC = 2
_NS = 16
_NW = _NC * _NS

_N = _BATCH * _SEQ_LEN
_IDX_MINOR = 128
_MAJOR = _N // _IDX_MINOR
_R_PER_W = _MAJOR // _NW

_ROWS = 5
_NBUF = 4
_N_CHUNKS = _R_PER_W // _ROWS
assert _N_CHUNKS % _NBUF == 0

_CHUNK_BYTES = _ROWS * _IDX_MINOR * _EMBED_DIM * 4


def _body(xt_hbm, table_hbm, out_hbm, idx_all, rows_v, gsem, wsem):
    wid = lax.axis_index("s") * _NC + lax.axis_index("c")
    row_base = wid * _R_PER_W

    pltpu.sync_copy(xt_hbm.at[pl.ds(row_base, _R_PER_W)], idx_all)

    def fire_gather(g, b):
        for j in range(_ROWS):
            pltpu.async_copy(
                table_hbm.at[idx_all.at[g * _ROWS + j]],
                rows_v.at[b].at[j],
                gsem.at[b],
            )

    def drain_gather(b):
        pltpu.make_async_copy(
            out_hbm.at[pl.ds(0, _ROWS)], rows_v.at[b], gsem.at[b]
        ).wait()

    def fire_write(g, b):
        pltpu.async_copy(
            rows_v.at[b],
            out_hbm.at[pl.ds(row_base + g * _ROWS, _ROWS)],
            wsem.at[b],
        )

    def drain_write(b):
        pltpu.make_async_copy(
            rows_v.at[b], out_hbm.at[pl.ds(0, _ROWS)], wsem.at[b]
        ).wait()

    for b in range(_NBUF):
        fire_gather(b, b)

    @pl.loop(0, _N_CHUNKS, step=_NBUF)
    def _group(g0):
        for b in range(_NBUF):
            g = g0 + b
            drain_gather(b)
            fire_write(g, b)

            @pl.when(g0 < _N_CHUNKS - _NBUF)
            def _refill():
                drain_write(b)
                fire_gather(g + _NBUF, b)

    for b in range(_NBUF):
        drain_write(b)


@jax.jit
def kernel(x, table):
    xt = jnp.transpose(x).reshape(_MAJOR, _IDX_MINOR).astype(jnp.int32)

    mesh = plsc.VectorSubcoreMesh(
        core_axis_name="c", subcore_axis_name="s",
        num_cores=_NC, num_subcores=_NS,
    )
    out = pl.kernel(
        _body,
        out_type=jax.ShapeDtypeStruct((_MAJOR, _IDX_MINOR, _EMBED_DIM), jnp.float32),
        mesh=mesh,
        scratch_types=[
            pltpu.VMEM((_R_PER_W, _IDX_MINOR), jnp.int32),
            pltpu.VMEM((_NBUF, _ROWS, _IDX_MINOR, _EMBED_DIM), jnp.float32),
            pltpu.SemaphoreType.DMA((_NBUF,)),
            pltpu.SemaphoreType.DMA((_NBUF,)),
        ],
        compiler_params=pltpu.CompilerParams(use_tc_tiling_on_sc=False),
    )(xt, table)
    return out.reshape(_SEQ_LEN, _BATCH, _EMBED_DIM)
